# baseline (speedup 1.0000x reference)
import functools

import jax
import jax.numpy as jnp
from jax import lax
from jax.experimental import pallas as pl
from jax.experimental.pallas import tpu as pltpu
from jax.experimental.pallas import tpu_sc as plsc

_D = 2048
_CH = 32


def _sc_gather(table, idx3):
    info = plsc.get_sparse_core_info()
    nc, ns = info.num_cores, info.num_subcores
    nw = nc * ns
    nch = idx3.shape[1]
    b = nw * nch * _CH
    mesh = plsc.VectorSubcoreMesh(core_axis_name="c", subcore_axis_name="s")

    @functools.partial(
        pl.kernel,
        mesh=mesh,
        out_type=jax.ShapeDtypeStruct((b, _D), jnp.float32),
        scratch_types=[
            pltpu.VMEM((nch, _CH), jnp.int32),
            pltpu.VMEM((_CH, _D), jnp.float32),
            pltpu.SemaphoreType.DMA,
        ],
    )
    def k(table_hbm, idx_hbm, out_hbm, idx_v, rows_v, sem):
        wid = lax.axis_index("s") * nc + lax.axis_index("c")
        base = wid * (nch * _CH)
        pltpu.sync_copy(idx_hbm.at[wid], idx_v)
        for j in range(nch):
            pltpu.async_copy(table_hbm.at[idx_v.at[j]], rows_v, sem).wait()
            pltpu.sync_copy(rows_v, out_hbm.at[pl.ds(base + j * _CH, _CH)])

    return k(table, idx3)


def _tc_assemble(cls_flat, prefix_flat, suffix_flat, b, br):
    pw = prefix_flat.shape[1]
    sw = suffix_flat.shape[1]
    ow = pw + _D + sw

    def body(cls_ref, pre_ref, suf_ref, out_ref):
        out_ref[:, 0:pw] = jnp.broadcast_to(pre_ref[...], (br, pw))
        out_ref[:, pw:pw + _D] = cls_ref[...]
        out_ref[:, pw + _D:ow] = jnp.broadcast_to(suf_ref[...], (br, sw))

    return pl.pallas_call(
        body,
        grid=(b // br,),
        in_specs=[
            pl.BlockSpec((br, _D), lambda i: (i, 0)),
            pl.BlockSpec(memory_space=pltpu.VMEM),
            pl.BlockSpec(memory_space=pltpu.VMEM),
        ],
        out_specs=pl.BlockSpec((br, ow), lambda i: (i, 0)),
        out_shape=jax.ShapeDtypeStruct((b, ow), jnp.float32),
        compiler_params=pltpu.CompilerParams(
            dimension_semantics=("arbitrary",),
        ),
    )(cls_flat, prefix_flat, suffix_flat)


def kernel(label, cls_ctx, token_prefix, token_suffix):
    b = label.shape[0]
    v = cls_ctx.shape[0]
    d = cls_ctx.shape[2]
    n_cls = cls_ctx.shape[1]
    n_pre = token_prefix.shape[1]
    n_suf = token_suffix.shape[1]
    seq = n_pre + n_cls + n_suf

    info = plsc.get_sparse_core_info()
    nw = info.num_cores * info.num_subcores
    nch = b // (nw * _CH)

    table = cls_ctx.reshape(v, n_cls * d)
    idx3 = label.astype(jnp.int32).reshape(nw, nch, _CH)
    cls_flat = _sc_gather(table, idx3)

    prefix_flat = token_prefix.reshape(1, n_pre * d)
    suffix_flat = token_suffix.reshape(1, n_suf * d)
    out_flat = _tc_assemble(cls_flat, prefix_flat, suffix_flat, b, br=64)
    return out_flat.reshape(b, seq, d)


# speedup vs baseline: 2.7472x; 2.7472x over previous
import functools

import jax
import jax.numpy as jnp
from jax import lax
from jax.experimental import pallas as pl
from jax.experimental.pallas import tpu as pltpu
from jax.experimental.pallas import tpu_sc as plsc

_CH = 32


def _sc_gather(table, idx3):
    _, c, d = table.shape
    info = plsc.get_sparse_core_info()
    nc, ns = info.num_cores, info.num_subcores
    nw = nc * ns
    nch = idx3.shape[1]
    b = nw * nch * _CH
    mesh = plsc.VectorSubcoreMesh(core_axis_name="c", subcore_axis_name="s")

    @functools.partial(
        pl.kernel,
        mesh=mesh,
        out_type=jax.ShapeDtypeStruct((b, c, d), jnp.float32),
        scratch_types=[
            pltpu.VMEM((nch, _CH), jnp.int32),
            pltpu.VMEM((_CH, c, d), jnp.float32),
            pltpu.SemaphoreType.DMA,
        ],
    )
    def k(table_hbm, idx_hbm, out_hbm, idx_v, rows_v, sem):
        wid = lax.axis_index("s") * nc + lax.axis_index("c")
        base = wid * (nch * _CH)
        pltpu.sync_copy(idx_hbm.at[wid], idx_v)
        for j in range(nch):
            pltpu.async_copy(table_hbm.at[idx_v.at[j]], rows_v, sem).wait()
            pltpu.sync_copy(rows_v, out_hbm.at[pl.ds(base + j * _CH, _CH)])

    return k(table, idx3)


def _tc_assemble(cls, prefix, suffix, br):
    b, c, d = cls.shape
    p = prefix.shape[1]
    s = suffix.shape[1]
    seq = p + c + s

    def body(cls_ref, pre_ref, suf_ref, out_ref):
        out_ref[:, 0:p, :] = jnp.broadcast_to(pre_ref[...], (br, p, d))
        out_ref[:, p:p + c, :] = cls_ref[...]
        out_ref[:, p + c:seq, :] = jnp.broadcast_to(suf_ref[...], (br, s, d))

    return pl.pallas_call(
        body,
        grid=(b // br,),
        in_specs=[
            pl.BlockSpec((br, c, d), lambda i: (i, 0, 0)),
            pl.BlockSpec(memory_space=pltpu.VMEM),
            pl.BlockSpec(memory_space=pltpu.VMEM),
        ],
        out_specs=pl.BlockSpec((br, seq, d), lambda i: (i, 0, 0)),
        out_shape=jax.ShapeDtypeStruct((b, seq, d), jnp.float32),
        compiler_params=pltpu.CompilerParams(
            dimension_semantics=("arbitrary",),
        ),
    )(cls, prefix, suffix)


def kernel(label, cls_ctx, token_prefix, token_suffix):
    b = label.shape[0]

    info = plsc.get_sparse_core_info()
    nw = info.num_cores * info.num_subcores
    nch = b // (nw * _CH)

    idx3 = label.astype(jnp.int32).reshape(nw, nch, _CH)
    cls = _sc_gather(cls_ctx, idx3)
    return _tc_assemble(cls, token_prefix, token_suffix, br=64)
